# baseline (speedup 1.0000x reference)
import functools

import jax
import jax.numpy as jnp
from jax import lax
from jax.experimental import pallas as pl
from jax.experimental.pallas import tpu as pltpu
from jax.experimental.pallas import tpu_sc as plsc

CLIP = 64
W_PHASES = 8


def _build_bands_kernel(table_ref, *out_refs, S, C, D, W):
    t0 = table_ref[0:1, :]
    tmid = table_ref[1 : 2 * C, :]
    tlast = table_ref[2 * C : 2 * C + 1, :]
    for k in range(W):
        e_ref = out_refs[k]
        e_ref[0 : k + S - C, :] = jnp.broadcast_to(t0, (k + S - C, D))
        e_ref[k + S - C : k + S - 1 + C, :] = tmid
        e_ref[k + S - 1 + C :, :] = jnp.broadcast_to(tlast, (2 * S + W - (k + S - 1 + C), D))


def _make_sc_window_kernel(S, D, NC, NS, W):
    n_rows = S // (NC * NS)
    mesh = plsc.VectorSubcoreMesh(core_axis_name="c", subcore_axis_name="s")

    @functools.partial(
        pl.kernel,
        out_type=jax.ShapeDtypeStruct((S, S, D), jnp.float32),
        mesh=mesh,
        scratch_types=[
            pltpu.SemaphoreType.DMA,
        ],
    )
    def sc_kernel(*refs):
        e_hbms = refs[:W]
        out_hbm = refs[W]
        sem = refs[W + 1]

        cid = lax.axis_index("c")
        sid = lax.axis_index("s")

        wid = sid * NC + cid
        base = wid * n_rows
        K = 8

        def mk_wait():
            return pltpu.make_async_copy(
                e_hbms[0].at[pl.ds(0, S), :], out_hbm.at[0], sem
            )

        def start_copy(t):
            r = base + t
            dst = out_hbm.at[r]
            for k in range(W):
                @pl.when(lax.rem(r + 1, W) == k)
                def _(k=k):
                    row0 = pl.multiple_of(S - 1 - r + k, W)
                    pltpu.make_async_copy(
                        e_hbms[k].at[pl.ds(row0, S), :], dst, sem
                    ).start()

        def body(t, _):
            @pl.when(t >= K)
            def _():
                mk_wait().wait()

            start_copy(t)
            return 0

        lax.fori_loop(0, n_rows, body, 0)

        def drain(k, _):
            mk_wait().wait()
            return 0

        lax.fori_loop(0, K, drain, 0)

    return sc_kernel


def _rel_pos_encoding(table, S, C, D, interpret=False):
    W = W_PHASES
    bands = pl.pallas_call(
        lambda t, *outs: _build_bands_kernel(t, *outs, S=S, C=C, D=D, W=W),
        in_specs=[pl.BlockSpec(memory_space=pltpu.VMEM)],
        out_specs=[pl.BlockSpec(memory_space=pltpu.VMEM) for _ in range(W)],
        out_shape=[jax.ShapeDtypeStruct((2 * S + W, D), table.dtype) for _ in range(W)],
        interpret=interpret,
    )(table)
    sc_kernel = _make_sc_window_kernel(S, D, 2, 16, W)
    return sc_kernel(*bands)


def kernel(x, encoding_matrix):
    S = x.shape[1]
    D = encoding_matrix.shape[1]
    return _rel_pos_encoding(encoding_matrix, S, CLIP, D)


# speedup vs baseline: 31.4267x; 31.4267x over previous
import functools

import jax
import jax.numpy as jnp
from jax import lax
from jax.experimental import pallas as pl
from jax.experimental.pallas import tpu as pltpu
from jax.experimental.pallas import tpu_sc as plsc

CLIP = 64


def _build_band_kernel(table_ref, e_ref, *, S, C, D):
    e_ref[0 : S - C, :] = jnp.broadcast_to(table_ref[0:1, :], (S - C, D))
    e_ref[S - C : S - 1 + C, :] = table_ref[1 : 2 * C, :]
    e_ref[S - 1 + C :, :] = jnp.broadcast_to(table_ref[2 * C : 2 * C + 1, :], (S - C + 1, D))


def _make_sc_window_kernel(S, D, NC, NS):
    n_rows = S // (NC * NS)
    CW = 512
    n_chunks = S // CW
    mesh = plsc.VectorSubcoreMesh(core_axis_name="c", subcore_axis_name="s")

    @functools.partial(
        pl.kernel,
        out_type=jax.ShapeDtypeStruct((S, S, D), jnp.float32),
        mesh=mesh,
        scratch_types=[
            pltpu.VMEM((CW + n_rows, D), jnp.float32),
            pltpu.SemaphoreType.DMA,
            pltpu.SemaphoreType.DMA,
        ],
    )
    def sc_kernel(e_hbm, out_hbm, buf, sem_in, sem):
        cid = lax.axis_index("c")
        sid = lax.axis_index("s")

        wid = sid * NC + cid
        base = wid * n_rows
        K = 8

        def mk_wait():
            return pltpu.make_async_copy(
                buf.at[pl.ds(0, CW), :], out_hbm.at[0, pl.ds(0, CW), :], sem
            )

        for c in range(n_chunks):
            src_lo = pl.multiple_of(S - n_rows - base + c * CW, 8)
            pltpu.make_async_copy(
                e_hbm.at[pl.ds(src_lo, CW + n_rows), :], buf, sem_in
            ).start()
            pltpu.make_async_copy(
                e_hbm.at[pl.ds(src_lo, CW + n_rows), :], buf, sem_in
            ).wait()

            def body(t, _):
                @pl.when(t >= K)
                def _():
                    mk_wait().wait()

                r = base + t
                pltpu.make_async_copy(
                    buf.at[pl.ds(n_rows - 1 - t, CW), :],
                    out_hbm.at[r, pl.ds(c * CW, CW), :],
                    sem,
                ).start()
                return 0

            lax.fori_loop(0, n_rows, body, 0)

            def drain(k, _):
                mk_wait().wait()
                return 0

            lax.fori_loop(0, K, drain, 0)

    return sc_kernel


def _rel_pos_encoding(table, S, C, D, interpret=False):
    band = pl.pallas_call(
        lambda t, e: _build_band_kernel(t, e, S=S, C=C, D=D),
        in_specs=[pl.BlockSpec(memory_space=pltpu.VMEM)],
        out_specs=pl.BlockSpec(memory_space=pltpu.VMEM),
        out_shape=jax.ShapeDtypeStruct((2 * S, D), table.dtype),
        interpret=interpret,
    )(table)
    sc_kernel = _make_sc_window_kernel(S, D, 2, 16)
    return sc_kernel(band)


def kernel(x, encoding_matrix):
    S = x.shape[1]
    D = encoding_matrix.shape[1]
    return _rel_pos_encoding(encoding_matrix, S, CLIP, D)
